# baseline (speedup 1.0000x reference)
import functools

import jax
import jax.numpy as jnp
from jax import lax
from jax.experimental import pallas as pl
from jax.experimental.pallas import tpu as pltpu
from jax.experimental.pallas import tpu_sc as plsc

N = 10000
E = 320000
D = 128

NC = 2
NS = 16
NW = NC * NS
L = 16

C = 128
NCHUNK = 80
NSTAGE = 8
E_PW = NCHUNK * C
E_PAD = NW * E_PW
DUMP = N
N_SH = 10240
N_DEG = 10240
N_DINV = 10240

_mesh = plsc.VectorSubcoreMesh(core_axis_name="c", subcore_axis_name="s")
_params = pltpu.CompilerParams(needs_layout_passes=False)


def _deg_body(col_hbm, ew_hbm, degp_hbm, cstage, wstage, wrows, zw, deg_sh,
              sem):
    c = lax.axis_index("c")
    s = lax.axis_index("s")
    wid = c * NS + s
    zero16f = jnp.zeros((L,), jnp.float32)

    def _zw(i, _):
        for g in range(D // L):
            wrows[i, pl.ds(g * L, L)] = zero16f
        return _
    lax.fori_loop(0, C, _zw, None)

    def _z(i, _):
        for g in range(D // L):
            zw[i, pl.ds(g * L, L)] = zero16f
        return _
    lax.fori_loop(0, 64, _z, None)
    for r in range(10):
        pltpu.sync_copy(zw, deg_sh.at[pl.ds(s * 640 + r * 64, 64)])
    plsc.subcore_barrier()

    for m in range(NCHUNK // NSTAGE):
        base = wid * NCHUNK + m * NSTAGE
        pltpu.sync_copy(col_hbm.at[pl.ds(base, NSTAGE)], cstage)
        pltpu.sync_copy(ew_hbm.at[pl.ds(base, NSTAGE)], wstage)

        def _chunk(k, _):
            def _grp(j, _g):
                v = wstage[k, pl.ds(j * L, L)]
                for l in range(L):
                    wrows[j * L + l, pl.ds(0, L)] = lax.broadcast(v[l], (L,))
                return _g
            lax.fori_loop(0, C // L, _grp, None)
            pltpu.sync_copy(wrows, deg_sh.at[cstage.at[k]], add=True)
            return _
        lax.fori_loop(0, NSTAGE, _chunk, None)
    plsc.subcore_barrier()

    for r in range(10):
        base = s * 640 + r * 64
        pltpu.sync_copy(deg_sh.at[pl.ds(base, 64)], zw)
        pltpu.sync_copy(zw, degp_hbm.at[pl.ds(c * N_DEG + base, 64)])


_deg_kernel = functools.partial(
    pl.kernel,
    out_type=jax.ShapeDtypeStruct((NC * N_DEG, D), jnp.float32),
    mesh=_mesh,
    compiler_params=_params,
    scratch_types=[
        pltpu.VMEM((NSTAGE, C), jnp.int32),
        pltpu.VMEM((NSTAGE, C), jnp.float32),
        pltpu.VMEM((C, D), jnp.float32),
        pltpu.VMEM((64, D), jnp.float32),
        pltpu.VMEM_SHARED((N_DEG, D), jnp.float32),
        pltpu.SemaphoreType.DMA,
    ],
)(_deg_body)


def _msg_body(row_hbm, col_hbm, ew_hbm, h_hbm, dinv_hbm, accp_hbm,
              rstage, cstage, wstage, dinv_v, rows_a, s_v, zbuf,
              out_sh, sem_a):
    c = lax.axis_index("c")
    s = lax.axis_index("s")
    wid = c * NS + s
    zero16f = jnp.zeros((L,), jnp.float32)
    m127 = jnp.full((L,), 127, jnp.int32)

    pltpu.sync_copy(dinv_hbm, dinv_v)

    def _z(i, _):
        for g in range(D // L):
            zbuf[i, pl.ds(g * L, L)] = zero16f
        return _
    lax.fori_loop(0, 16, _z, None)
    for r in range(40):
        pltpu.sync_copy(zbuf, out_sh.at[pl.ds(s * 640 + r * 16, 16)])
    plsc.subcore_barrier()

    for m in range(NCHUNK // NSTAGE):
        base = wid * NCHUNK + m * NSTAGE
        pltpu.sync_copy(row_hbm.at[pl.ds(base, NSTAGE)], rstage)
        pltpu.sync_copy(col_hbm.at[pl.ds(base, NSTAGE)], cstage)
        pltpu.sync_copy(ew_hbm.at[pl.ds(base, NSTAGE)], wstage)

        def _chunk(k, _):
            pltpu.async_copy(h_hbm.at[rstage.at[k]], rows_a, sem_a).wait()
            for j in range(C // L):
                r16 = rstage[k, pl.ds(j * L, L)]
                c16 = cstage[k, pl.ds(j * L, L)]
                dr = plsc.load_gather(
                    dinv_v, [lax.shift_right_logical(r16, 7), r16 & m127])
                dc = plsc.load_gather(
                    dinv_v, [lax.shift_right_logical(c16, 7), c16 & m127])
                s_v[pl.ds(j * L, L)] = wstage[k, pl.ds(j * L, L)] * dr * dc

            def _scale(j, _s):
                v = s_v[pl.ds(j * L, L)]
                for l in range(L):
                    sc16 = lax.broadcast(v[l], (L,))
                    e = j * L + l
                    for g in range(D // L):
                        rows_a[e, pl.ds(g * L, L)] = (
                            rows_a[e, pl.ds(g * L, L)] * sc16)
                return _s
            lax.fori_loop(0, C // L, _scale, None)
            pltpu.sync_copy(rows_a, out_sh.at[cstage.at[k]], add=True)
            return _
        lax.fori_loop(0, NSTAGE, _chunk, None)
    plsc.subcore_barrier()

    for r in range(40):
        base = s * 640 + r * 16
        pltpu.sync_copy(out_sh.at[pl.ds(base, 16)], zbuf)
        pltpu.sync_copy(zbuf, accp_hbm.at[c, pl.ds(base, 16)])


_msg_kernel = functools.partial(
    pl.kernel,
    out_type=jax.ShapeDtypeStruct((NC, N_SH, D), jnp.float32),
    mesh=_mesh,
    compiler_params=_params,
    scratch_types=[
        pltpu.VMEM((NSTAGE, C), jnp.int32),
        pltpu.VMEM((NSTAGE, C), jnp.int32),
        pltpu.VMEM((NSTAGE, C), jnp.float32),
        pltpu.VMEM((N_DINV // D, D), jnp.float32),
        pltpu.VMEM((C, D), jnp.float32),
        pltpu.VMEM((C,), jnp.float32),
        pltpu.VMEM((16, D), jnp.float32),
        pltpu.VMEM_SHARED((N_SH, D), jnp.float32),
        pltpu.SemaphoreType.DMA,
    ],
)(_msg_body)


def _mm_body(x_ref, w_ref, o_ref):
    o_ref[...] = jnp.dot(x_ref[...], w_ref[...],
                         preferred_element_type=jnp.float32)


def _fin_body(a_ref, h_ref, d_ref, b_ref, o_ref):
    o_ref[...] = jnp.maximum(
        a_ref[0] + a_ref[1] + h_ref[...] * d_ref[...] + b_ref[...], 0.0)


def kernel(x, edge_index, edge_weight, W, b):
    ei = edge_index.astype(jnp.int32)
    row = jnp.concatenate(
        [ei[0], jnp.zeros((E_PAD - E,), jnp.int32)]).reshape(E_PAD // C, C)
    col = jnp.concatenate(
        [ei[1], jnp.full((E_PAD - E,), DUMP, jnp.int32)]).reshape(E_PAD // C, C)
    ew = jnp.concatenate(
        [edge_weight, jnp.zeros((E_PAD - E,), jnp.float32)]
    ).reshape(E_PAD // C, C)

    degp = _deg_kernel(col, ew)
    deg = degp[:N, 0] + degp[N_DEG:N_DEG + N, 0] + 1.0
    dinv = lax.rsqrt(deg)
    dinv_p = jnp.pad(dinv, (0, N_DINV - N)).reshape(N_DINV // D, D)

    h = pl.pallas_call(
        _mm_body,
        grid=(10,),
        in_specs=[
            pl.BlockSpec((N // 10, D), lambda i: (i, 0)),
            pl.BlockSpec((D, D), lambda i: (0, 0)),
        ],
        out_specs=pl.BlockSpec((N // 10, D), lambda i: (i, 0)),
        out_shape=jax.ShapeDtypeStruct((N, D), jnp.float32),
    )(x, W)

    accp = _msg_kernel(row, col, ew, h, dinv_p)

    dinv2 = (dinv * dinv)[:, None]
    out = pl.pallas_call(
        _fin_body,
        grid=(10,),
        in_specs=[
            pl.BlockSpec((NC, N // 10, D), lambda i: (0, i, 0)),
            pl.BlockSpec((N // 10, D), lambda i: (i, 0)),
            pl.BlockSpec((N // 10, 1), lambda i: (i, 0)),
            pl.BlockSpec((1, D), lambda i: (0, 0)),
        ],
        out_specs=pl.BlockSpec((N // 10, D), lambda i: (i, 0)),
        out_shape=jax.ShapeDtypeStruct((N, D), jnp.float32),
    )(accp, h, dinv2, b[None, :])
    return out


# speedup vs baseline: 1.0248x; 1.0248x over previous
import functools

import jax
import jax.numpy as jnp
from jax import lax
from jax.experimental import pallas as pl
from jax.experimental.pallas import tpu as pltpu
from jax.experimental.pallas import tpu_sc as plsc

N = 10000
E = 320000
D = 128

NC = 2
NS = 16
NW = NC * NS
L = 16

C = 128
NCHUNK = 80
NSTAGE = 16
E_PW = NCHUNK * C
E_PAD = NW * E_PW
DUMP = N
N_SH = 10240
N_DEG = 10240
N_DINV = 10240

_mesh = plsc.VectorSubcoreMesh(core_axis_name="c", subcore_axis_name="s")
_params = pltpu.CompilerParams(needs_layout_passes=False)


def _deg_body(col_hbm, ew_hbm, degp_hbm, cstage, wstage, wrows, zw, deg_sh,
              sem):
    c = lax.axis_index("c")
    s = lax.axis_index("s")
    wid = c * NS + s
    zero16f = jnp.zeros((L,), jnp.float32)

    def _zw(i, _):
        for g in range(D // L):
            wrows[i, pl.ds(g * L, L)] = zero16f
        return _
    lax.fori_loop(0, C, _zw, None)

    def _z(i, _):
        for g in range(D // L):
            zw[i, pl.ds(g * L, L)] = zero16f
        return _
    lax.fori_loop(0, 64, _z, None)
    for r in range(10):
        pltpu.sync_copy(zw, deg_sh.at[pl.ds(s * 640 + r * 64, 64)])
    plsc.subcore_barrier()

    for m in range(NCHUNK // NSTAGE):
        base = wid * NCHUNK + m * NSTAGE
        pltpu.sync_copy(col_hbm.at[pl.ds(base, NSTAGE)], cstage)
        pltpu.sync_copy(ew_hbm.at[pl.ds(base, NSTAGE)], wstage)

        def _chunk(k, _):
            def _grp(j, _g):
                v = wstage[k, pl.ds(j * L, L)]
                for l in range(L):
                    wrows[j * L + l, pl.ds(0, L)] = lax.broadcast(v[l], (L,))
                return _g
            lax.fori_loop(0, C // L, _grp, None)
            pltpu.sync_copy(wrows, deg_sh.at[cstage.at[k]], add=True)
            return _
        lax.fori_loop(0, NSTAGE, _chunk, None)
    plsc.subcore_barrier()

    for r in range(10):
        base = s * 640 + r * 64
        pltpu.sync_copy(deg_sh.at[pl.ds(base, 64)], zw)
        pltpu.sync_copy(zw, degp_hbm.at[pl.ds(c * N_DEG + base, 64)])


_deg_kernel = functools.partial(
    pl.kernel,
    out_type=jax.ShapeDtypeStruct((NC * N_DEG, D), jnp.float32),
    mesh=_mesh,
    compiler_params=_params,
    scratch_types=[
        pltpu.VMEM((NSTAGE, C), jnp.int32),
        pltpu.VMEM((NSTAGE, C), jnp.float32),
        pltpu.VMEM((C, D), jnp.float32),
        pltpu.VMEM((64, D), jnp.float32),
        pltpu.VMEM_SHARED((N_DEG, D), jnp.float32),
        pltpu.SemaphoreType.DMA,
    ],
)(_deg_body)


def _msg_body(row_hbm, col_hbm, ew_hbm, h_hbm, dinv_hbm, accp_hbm,
              rstage, cstage, wstage, dinv_v, rows_a, s_v, zbuf,
              out_sh, sem_a):
    c = lax.axis_index("c")
    s = lax.axis_index("s")
    wid = c * NS + s
    zero16f = jnp.zeros((L,), jnp.float32)
    m127 = jnp.full((L,), 127, jnp.int32)

    pltpu.sync_copy(dinv_hbm, dinv_v)

    def _z(i, _):
        for g in range(D // L):
            zbuf[i, pl.ds(g * L, L)] = zero16f
        return _
    lax.fori_loop(0, 32, _z, None)
    for r in range(20):
        pltpu.sync_copy(zbuf, out_sh.at[pl.ds(s * 640 + r * 32, 32)])
    plsc.subcore_barrier()

    for m in range(NCHUNK // NSTAGE):
        base = wid * NCHUNK + m * NSTAGE
        pltpu.sync_copy(row_hbm.at[pl.ds(base, NSTAGE)], rstage)
        pltpu.sync_copy(col_hbm.at[pl.ds(base, NSTAGE)], cstage)
        pltpu.sync_copy(ew_hbm.at[pl.ds(base, NSTAGE)], wstage)

        def _chunk(k, _):
            pltpu.async_copy(h_hbm.at[rstage.at[k]], rows_a, sem_a).wait()
            for j in range(C // L):
                r16 = rstage[k, pl.ds(j * L, L)]
                c16 = cstage[k, pl.ds(j * L, L)]
                dr = plsc.load_gather(
                    dinv_v, [lax.shift_right_logical(r16, 7), r16 & m127])
                dc = plsc.load_gather(
                    dinv_v, [lax.shift_right_logical(c16, 7), c16 & m127])
                s_v[pl.ds(j * L, L)] = wstage[k, pl.ds(j * L, L)] * dr * dc

            def _scale(j, _s):
                v = s_v[pl.ds(j * L, L)]
                for l in range(L):
                    sc16 = lax.broadcast(v[l], (L,))
                    e = j * L + l
                    for g in range(D // L):
                        rows_a[e, pl.ds(g * L, L)] = (
                            rows_a[e, pl.ds(g * L, L)] * sc16)
                return _s
            lax.fori_loop(0, C // L, _scale, None)
            pltpu.sync_copy(rows_a, out_sh.at[cstage.at[k]], add=True)
            return _
        lax.fori_loop(0, NSTAGE, _chunk, None)
    plsc.subcore_barrier()

    for r in range(20):
        base = s * 640 + r * 32
        pltpu.sync_copy(out_sh.at[pl.ds(base, 32)], zbuf)
        pltpu.sync_copy(zbuf, accp_hbm.at[c, pl.ds(base, 32)])


_msg_kernel = functools.partial(
    pl.kernel,
    out_type=jax.ShapeDtypeStruct((NC, N_SH, D), jnp.float32),
    mesh=_mesh,
    compiler_params=_params,
    scratch_types=[
        pltpu.VMEM((NSTAGE, C), jnp.int32),
        pltpu.VMEM((NSTAGE, C), jnp.int32),
        pltpu.VMEM((NSTAGE, C), jnp.float32),
        pltpu.VMEM((N_DINV // D, D), jnp.float32),
        pltpu.VMEM((C, D), jnp.float32),
        pltpu.VMEM((C,), jnp.float32),
        pltpu.VMEM((32, D), jnp.float32),
        pltpu.VMEM_SHARED((N_SH, D), jnp.float32),
        pltpu.SemaphoreType.DMA,
    ],
)(_msg_body)


def _mm_body(x_ref, w_ref, o_ref):
    o_ref[...] = jnp.dot(x_ref[...], w_ref[...],
                         preferred_element_type=jnp.float32)


def _fin_body(a_ref, h_ref, d_ref, b_ref, o_ref):
    o_ref[...] = jnp.maximum(
        a_ref[0] + a_ref[1] + h_ref[...] * d_ref[...] + b_ref[...], 0.0)


def kernel(x, edge_index, edge_weight, W, b):
    ei = edge_index.astype(jnp.int32)
    row = jnp.concatenate(
        [ei[0], jnp.zeros((E_PAD - E,), jnp.int32)]).reshape(E_PAD // C, C)
    col = jnp.concatenate(
        [ei[1], jnp.full((E_PAD - E,), DUMP, jnp.int32)]).reshape(E_PAD // C, C)
    ew = jnp.concatenate(
        [edge_weight, jnp.zeros((E_PAD - E,), jnp.float32)]
    ).reshape(E_PAD // C, C)

    degp = _deg_kernel(col, ew)
    deg = degp[:N, 0] + degp[N_DEG:N_DEG + N, 0] + 1.0
    dinv = lax.rsqrt(deg)
    dinv_p = jnp.pad(dinv, (0, N_DINV - N)).reshape(N_DINV // D, D)

    h = pl.pallas_call(
        _mm_body,
        grid=(10,),
        in_specs=[
            pl.BlockSpec((N // 10, D), lambda i: (i, 0)),
            pl.BlockSpec((D, D), lambda i: (0, 0)),
        ],
        out_specs=pl.BlockSpec((N // 10, D), lambda i: (i, 0)),
        out_shape=jax.ShapeDtypeStruct((N, D), jnp.float32),
    )(x, W)

    accp = _msg_kernel(row, col, ew, h, dinv_p)

    dinv2 = (dinv * dinv)[:, None]
    out = pl.pallas_call(
        _fin_body,
        grid=(10,),
        in_specs=[
            pl.BlockSpec((NC, N // 10, D), lambda i: (0, i, 0)),
            pl.BlockSpec((N // 10, D), lambda i: (i, 0)),
            pl.BlockSpec((N // 10, 1), lambda i: (i, 0)),
            pl.BlockSpec((1, D), lambda i: (0, 0)),
        ],
        out_specs=pl.BlockSpec((N // 10, D), lambda i: (i, 0)),
        out_shape=jax.ShapeDtypeStruct((N, D), jnp.float32),
    )(accp, h, dinv2, b[None, :])
    return out
